# baseline (speedup 1.0000x reference)
import functools

import jax
import jax.numpy as jnp
from jax import lax
from jax.experimental import pallas as pl
from jax.experimental.pallas import tpu as pltpu
from jax.experimental.pallas import tpu_sc as plsc

BATCH = 16384
NF = 26
VOCAB = 100000
D = 32
ROWS = BATCH * NF
NC = 2
NS = 16
NW = NC * NS
RPW = ROWS // NW
IDXW = 128
IDXROWS = ROWS // IDXW
IRPW = IDXROWS // NW
CH = 1664
NCH = RPW // CH
LANES = 16

_mesh = plsc.VectorSubcoreMesh(core_axis_name="c", subcore_axis_name="s")


@functools.partial(
    pl.kernel,
    mesh=_mesh,
    out_type=jax.ShapeDtypeStruct((ROWS, D), jnp.float32),
    compiler_params=pltpu.CompilerParams(use_tc_tiling_on_sc=False),
    scratch_types=[
        pltpu.VMEM((IRPW, IDXW), jnp.int32),
        pltpu.VMEM((RPW,), jnp.int32),
        pltpu.VMEM((CH, D), jnp.float32),
        pltpu.SemaphoreType.DMA,
    ],
)
def _emb_gather(idx_hbm, table_hbm, out_hbm, idx2_v, idx_v, rows_v, sem):
    wid = lax.axis_index("s") * NC + lax.axis_index("c")
    base = wid * RPW
    pltpu.sync_copy(idx_hbm.at[pl.ds(wid * IRPW, IRPW)], idx2_v)

    def flatten(r, carry):
        for k in range(IDXW // LANES):
            idx_v[pl.ds(r * IDXW + k * LANES, LANES)] = idx2_v[
                r, pl.ds(k * LANES, LANES)
            ]
        return carry

    lax.fori_loop(0, IRPW, flatten, 0)

    for c in range(NCH):
        pltpu.async_copy(
            table_hbm.at[idx_v.at[pl.ds(c * CH, CH)]], rows_v, sem
        ).wait()
        pltpu.sync_copy(rows_v, out_hbm.at[pl.ds(base + c * CH, CH)])


def kernel(indices, tables):
    offs = (jnp.arange(NF, dtype=jnp.int32) * VOCAB)[None, :]
    gidx = (indices.astype(jnp.int32) + offs).reshape(IDXROWS, IDXW)
    flat_tables = tables.reshape(NF * VOCAB, D)
    out = _emb_gather(gidx, flat_tables)
    return out.reshape(BATCH, NF * D)


# speedup vs baseline: 1.0033x; 1.0033x over previous
import functools

import jax
import jax.numpy as jnp
from jax import lax
from jax.experimental import pallas as pl
from jax.experimental.pallas import tpu as pltpu
from jax.experimental.pallas import tpu_sc as plsc

BATCH = 16384
NF = 26
VOCAB = 100000
D = 32
ROWS = BATCH * NF
NC = 2
NS = 16
NW = NC * NS
RPW = ROWS // NW
CH = 1664
NCH = RPW // CH
LANES = 16
BW = 128
NBT = BATCH // BW
BT_PER_W = NBT // NW
BLK = NF * BW
INV13 = -991146299

_mesh = plsc.VectorSubcoreMesh(core_axis_name="c", subcore_axis_name="s")


@functools.partial(
    pl.kernel,
    mesh=_mesh,
    out_type=jax.ShapeDtypeStruct((ROWS,), jnp.int32),
    compiler_params=pltpu.CompilerParams(
        use_tc_tiling_on_sc=True, needs_layout_passes=False
    ),
    scratch_types=[
        pltpu.VMEM((32, BW), jnp.int32),
        pltpu.VMEM((BLK,), jnp.int32),
    ],
)
def _idx_prep(idxt_hbm, tail_hbm, gidx_hbm, slab_v, buf_v):
    wid = lax.axis_index("s") * NC + lax.axis_index("c")
    lane = lax.iota(jnp.int32, LANES)
    fpats = tuple(lax.rem(k * LANES + lane, NF) for k in range(13))

    for bt_i in range(BT_PER_W):
        bt = wid * BT_PER_W + bt_i
        pltpu.sync_copy(
            idxt_hbm.at[pl.ds(0, 24), pl.ds(bt * BW, BW)],
            slab_v.at[pl.ds(0, 24)],
        )
        pltpu.sync_copy(
            tail_hbm.at[:, pl.ds(bt * BW, BW)], slab_v.at[pl.ds(24, 8)]
        )

        def body(g, carry):
            for kk in range(13):
                k = g * 13 + kk
                f = carry[kk]
                p = k * LANES + lane
                j = lax.shift_right_logical(p - f, 1) * jnp.int32(INV13)
                v = plsc.load_gather(slab_v, [f, j])
                buf_v[pl.ds(k * LANES, LANES)] = v + f * VOCAB
            return carry

        lax.fori_loop(0, 16, body, fpats)
        pltpu.sync_copy(buf_v, gidx_hbm.at[pl.ds(bt * BLK, BLK)])


@functools.partial(
    pl.kernel,
    mesh=_mesh,
    out_type=jax.ShapeDtypeStruct((ROWS, D), jnp.float32),
    compiler_params=pltpu.CompilerParams(use_tc_tiling_on_sc=False),
    scratch_types=[
        pltpu.VMEM((RPW,), jnp.int32),
        pltpu.VMEM((CH, D), jnp.float32),
        pltpu.SemaphoreType.DMA,
    ],
)
def _emb_gather(idx_hbm, table_hbm, out_hbm, idx_v, rows_v, sem):
    wid = lax.axis_index("s") * NC + lax.axis_index("c")
    base = wid * RPW
    pltpu.sync_copy(idx_hbm.at[pl.ds(base, RPW)], idx_v)

    for c in range(NCH):
        pltpu.async_copy(
            table_hbm.at[idx_v.at[pl.ds(c * CH, CH)]], rows_v, sem
        ).wait()
        pltpu.sync_copy(rows_v, out_hbm.at[pl.ds(base + c * CH, CH)])


def kernel(indices, tables):
    idxt = jnp.swapaxes(indices.astype(jnp.int32), 0, 1)
    tail = jnp.pad(idxt[24:26], ((0, 6), (0, 0)))
    gidx = _idx_prep(idxt, tail)
    flat_tables = tables.reshape(NF * VOCAB, D)
    out = _emb_gather(gidx, flat_tables)
    return out.reshape(BATCH, NF * D)
